# Initial kernel scaffold; baseline (speedup 1.0000x reference)
import jax
import jax.numpy as jnp
from jax.experimental import pallas as pl


def kernel(x, edge_index, poly_conn, sqrt_deg, qkv_weight, qkv_bias, conn_lin1_w, conn_lin1_b, Wscore, conn_lin2_w, conn_lin2_b, deg_coef, ffn1_w, ffn1_b, ffn2_w, ffn2_b):
    raise NotImplementedError("write your pallas kernel here")


# speedup vs baseline: 15.3842x; 15.3842x over previous
import functools

import jax
import jax.numpy as jnp
from jax import lax
from jax.experimental import pallas as pl


_CLAMP = 5.0


def _pick_block(total, target, mult):
    best = None
    d = mult
    while d <= target:
        if total % d == 0:
            best = d
        d += mult
    if best is None:
        best = total
    return best


def _qkv_body(x_ref, wt_ref, b_ref, o_ref):
    o_ref[...] = (
        jnp.dot(x_ref[...], wt_ref[...], preferred_element_type=jnp.float32)
        + b_ref[...]
    )


def _tc_qkv(x, qkv_weight, qkv_bias):
    n, d = x.shape
    bn = _pick_block(n, 2048, 8)
    grid = n // bn
    return pl.pallas_call(
        _qkv_body,
        grid=(grid,),
        in_specs=[
            pl.BlockSpec((bn, d), lambda i: (i, 0)),
            pl.BlockSpec((d, 3 * d), lambda i: (0, 0)),
            pl.BlockSpec((1, 3 * d), lambda i: (0, 0)),
        ],
        out_specs=pl.BlockSpec((bn, 3 * d), lambda i: (i, 0)),
        out_shape=jax.ShapeDtypeStruct((n, 3 * d), jnp.float32),
    )(x, qkv_weight.T, qkv_bias.reshape(1, -1))


def _edge_a_body(g_ref, poly_ref, w1t_ref, b1_ref, w2t_ref, b2_ref, wsc_ref,
                 conn2_ref, wpad_ref, stats_ref):
    i = pl.program_id(0)
    poly = poly_ref[...]
    d = poly.shape[1]
    eh = (
        jnp.dot(poly, w1t_ref[...], preferred_element_type=jnp.float32)
        + b1_ref[...]
    )
    ew = eh[:, :d]
    ebias = eh[:, d:]
    c = g_ref[...] * ew
    c = jnp.sign(c) * jnp.sqrt(jnp.abs(c)) + ebias
    conn = jnp.maximum(c, 0.0)
    conn2 = (
        jnp.dot(conn, w2t_ref[...], preferred_element_type=jnp.float32)
        + b2_ref[...]
        + poly
    )
    conn2_ref[...] = conn2
    s = jnp.dot(conn, wsc_ref[...], preferred_element_type=jnp.float32)
    wpad_ref[...] = jnp.exp(jnp.clip(s, -_CLAMP, _CLAMP))

    s1 = jnp.sum(conn2, axis=0)
    s2 = jnp.sum(conn2 * conn2, axis=0)
    blk = jnp.concatenate(
        [s1[None, :], s2[None, :], jnp.zeros((6, d), jnp.float32)], axis=0
    )

    @pl.when(i == 0)
    def _():
        stats_ref[...] = jnp.zeros_like(stats_ref)

    stats_ref[...] += blk


def _tc_edge_a(g, poly_conn, conn_lin1_w, conn_lin1_b, conn_lin2_w,
               conn_lin2_b, wsc2):
    e, d = poly_conn.shape
    be = _pick_block(e, 4096, 128)
    grid = e // be
    return pl.pallas_call(
        _edge_a_body,
        grid=(grid,),
        in_specs=[
            pl.BlockSpec((be, d), lambda i: (i, 0)),
            pl.BlockSpec((be, d), lambda i: (i, 0)),
            pl.BlockSpec((d, 2 * d), lambda i: (0, 0)),
            pl.BlockSpec((1, 2 * d), lambda i: (0, 0)),
            pl.BlockSpec((d, d), lambda i: (0, 0)),
            pl.BlockSpec((1, d), lambda i: (0, 0)),
            pl.BlockSpec((d, 16), lambda i: (0, 0)),
        ],
        out_specs=[
            pl.BlockSpec((be, d), lambda i: (i, 0)),
            pl.BlockSpec((be, 16), lambda i: (i, 0)),
            pl.BlockSpec((8, d), lambda i: (0, 0)),
        ],
        out_shape=[
            jax.ShapeDtypeStruct((e, d), jnp.float32),
            jax.ShapeDtypeStruct((e, 16), jnp.float32),
            jax.ShapeDtypeStruct((8, d), jnp.float32),
        ],
    )(g, poly_conn, conn_lin1_w.T, conn_lin1_b.reshape(1, -1),
      conn_lin2_w.T, conn_lin2_b.reshape(1, -1), wsc2)


def _node1_body(p0_ref, p1_ref, c0_ref, c1_ref, x_ref, sd_ref, dc0_ref,
                dc1_ref, hres_ref, stats_ref):
    i = pl.program_id(0)
    d = x_ref.shape[1]
    h = d // 16
    tbl = p0_ref[...] + p1_ref[...]
    naggs = tbl[:, :d] + c0_ref[...] + c1_ref[...]
    ssum = tbl[:, d:d + h]
    f_idx = lax.broadcasted_iota(jnp.int32, (h, d), 1)
    h_idx = lax.broadcasted_iota(jnp.int32, (h, d), 0)
    expand = (f_idx // 16 == h_idx).astype(jnp.float32)
    den = jnp.dot(ssum, expand, preferred_element_type=jnp.float32)
    agg = naggs / (den + 1e-16)
    nh = agg * (dc0_ref[...] + sd_ref[...] * dc1_ref[...])
    hres = nh + x_ref[...]
    hres_ref[...] = hres

    s1 = jnp.sum(hres, axis=0)
    s2 = jnp.sum(hres * hres, axis=0)
    blk = jnp.concatenate(
        [s1[None, :], s2[None, :], jnp.zeros((6, d), jnp.float32)], axis=0
    )

    @pl.when(i == 0)
    def _():
        stats_ref[...] = jnp.zeros_like(stats_ref)

    stats_ref[...] += blk


def _tc_node1(p0, p1, c0, c1, x, sqrt_deg, dc0, dc1):
    n, d = x.shape
    bn = _pick_block(n, 2048, 8)
    grid = n // bn
    return pl.pallas_call(
        _node1_body,
        grid=(grid,),
        in_specs=[
            pl.BlockSpec((bn, d + 16), lambda i: (i, 0)),
            pl.BlockSpec((bn, d + 16), lambda i: (i, 0)),
            pl.BlockSpec((bn, d), lambda i: (i, 0)),
            pl.BlockSpec((bn, d), lambda i: (i, 0)),
            pl.BlockSpec((bn, d), lambda i: (i, 0)),
            pl.BlockSpec((bn, 1), lambda i: (i, 0)),
            pl.BlockSpec((1, d), lambda i: (0, 0)),
            pl.BlockSpec((1, d), lambda i: (0, 0)),
        ],
        out_specs=[
            pl.BlockSpec((bn, d), lambda i: (i, 0)),
            pl.BlockSpec((8, d), lambda i: (0, 0)),
        ],
        out_shape=[
            jax.ShapeDtypeStruct((n, d), jnp.float32),
            jax.ShapeDtypeStruct((8, d), jnp.float32),
        ],
    )(p0, p1, c0, c1, x, sqrt_deg, dc0, dc1)


def _node2_body(hres_ref, mu_ref, iv_ref, f1t_ref, fb1_ref, f2t_ref, fb2_ref,
                t2_ref, stats_ref):
    i = pl.program_id(0)
    d = hres_ref.shape[1]
    hres = hres_ref[...]
    t = (hres - mu_ref[...]) * iv_ref[...]
    t1 = jnp.maximum(
        jnp.dot(t, f1t_ref[...], preferred_element_type=jnp.float32)
        + fb1_ref[...],
        0.0,
    )
    t2 = (
        jnp.dot(t1, f2t_ref[...], preferred_element_type=jnp.float32)
        + fb2_ref[...]
        + hres
    )
    t2_ref[...] = t2

    s1 = jnp.sum(t2, axis=0)
    s2 = jnp.sum(t2 * t2, axis=0)
    blk = jnp.concatenate(
        [s1[None, :], s2[None, :], jnp.zeros((6, d), jnp.float32)], axis=0
    )

    @pl.when(i == 0)
    def _():
        stats_ref[...] = jnp.zeros_like(stats_ref)

    stats_ref[...] += blk


def _tc_node2(hres, mu, iv, ffn1_w, ffn1_b, ffn2_w, ffn2_b):
    n, d = hres.shape
    bn = _pick_block(n, 2048, 8)
    grid = n // bn
    return pl.pallas_call(
        _node2_body,
        grid=(grid,),
        in_specs=[
            pl.BlockSpec((bn, d), lambda i: (i, 0)),
            pl.BlockSpec((1, d), lambda i: (0, 0)),
            pl.BlockSpec((1, d), lambda i: (0, 0)),
            pl.BlockSpec((d, 2 * d), lambda i: (0, 0)),
            pl.BlockSpec((1, 2 * d), lambda i: (0, 0)),
            pl.BlockSpec((2 * d, d), lambda i: (0, 0)),
            pl.BlockSpec((1, d), lambda i: (0, 0)),
        ],
        out_specs=[
            pl.BlockSpec((bn, d), lambda i: (i, 0)),
            pl.BlockSpec((8, d), lambda i: (0, 0)),
        ],
        out_shape=[
            jax.ShapeDtypeStruct((n, d), jnp.float32),
            jax.ShapeDtypeStruct((8, d), jnp.float32),
        ],
    )(hres, mu, iv, ffn1_w.T, ffn1_b.reshape(1, -1), ffn2_w.T,
      ffn2_b.reshape(1, -1))


def _node3_body(t2_ref, mu_ref, iv_ref, o_ref):
    o_ref[...] = (t2_ref[...] - mu_ref[...]) * iv_ref[...]


def _tc_node3(t2, mu, iv):
    n, d = t2.shape
    bn = _pick_block(n, 2048, 8)
    grid = n // bn
    return pl.pallas_call(
        _node3_body,
        grid=(grid,),
        in_specs=[
            pl.BlockSpec((bn, d), lambda i: (i, 0)),
            pl.BlockSpec((1, d), lambda i: (0, 0)),
            pl.BlockSpec((1, d), lambda i: (0, 0)),
        ],
        out_specs=pl.BlockSpec((bn, d), lambda i: (i, 0)),
        out_shape=jax.ShapeDtypeStruct((n, d), jnp.float32),
    )(t2, mu, iv)


def _stats_to_mu_inv(stats, count):
    s1 = stats[0]
    s2 = stats[1]
    mu = s1 / count
    var = s2 / count - mu * mu
    iv = 1.0 / jnp.sqrt(var + 1e-5)
    return mu.reshape(1, -1), iv.reshape(1, -1)


def kernel(x, edge_index, poly_conn, sqrt_deg, qkv_weight, qkv_bias,
           conn_lin1_w, conn_lin1_b, Wscore, conn_lin2_w, conn_lin2_b,
           deg_coef, ffn1_w, ffn1_b, ffn2_w, ffn2_b):
    n, d = x.shape
    e = poly_conn.shape[0]
    h = d // 16

    dst = edge_index[0]
    src = edge_index[1]

    wsq = Wscore[:, :, 0]
    sel = jnp.zeros((h, 16), jnp.float32).at[:, :h].set(jnp.eye(h))
    wsc2 = jnp.einsum('dh,hc->hdc', wsq, sel).reshape(d, 16)

    qkv = _tc_qkv(x, qkv_weight, qkv_bias)
    qh = qkv[:, :d]
    kh = qkv[:, d:2 * d]
    vh = qkv[:, 2 * d:]

    g = jnp.take(qh, dst, axis=0) + jnp.take(kh, src, axis=0)

    conn2raw, wpad, estats = _tc_edge_a(
        g, poly_conn, conn_lin1_w, conn_lin1_b, conn_lin2_w, conn_lin2_b,
        wsc2)
    emu, eiv = _stats_to_mu_inv(estats, float(e))

    w8 = wpad[:, :h]
    conn2n = jnp.maximum((conn2raw - emu) * eiv, 0.0)
    vsrc = jnp.take(vh, src, axis=0)
    nagg = jax.ops.segment_sum(
        (vsrc.reshape(e, h, 16) * w8[:, :, None]).reshape(e, d), dst,
        num_segments=n)
    cagg = jax.ops.segment_sum(
        (conn2n.reshape(e, h, 16) * w8[:, :, None]).reshape(e, d), dst,
        num_segments=n)
    ssum_pad = jax.ops.segment_sum(wpad, dst, num_segments=n)
    p0 = jnp.concatenate([nagg, ssum_pad], axis=1)
    p1 = jnp.zeros_like(p0)
    c0 = cagg
    c1 = jnp.zeros_like(c0)

    dc0 = deg_coef[0, :, 0].reshape(1, d)
    dc1 = deg_coef[0, :, 1].reshape(1, d)
    hres, nstats1 = _tc_node1(p0, p1, c0, c1, x, sqrt_deg, dc0, dc1)
    mu1, iv1 = _stats_to_mu_inv(nstats1, float(n))
    t2, nstats2 = _tc_node2(hres, mu1, iv1, ffn1_w, ffn1_b, ffn2_w, ffn2_b)
    mu2, iv2 = _stats_to_mu_inv(nstats2, float(n))
    nh = _tc_node3(t2, mu2, iv2)
    return nh, conn2n


# speedup vs baseline: 58.7803x; 3.8208x over previous
import functools

import jax
import jax.numpy as jnp
from jax import lax
from jax.experimental import pallas as pl
from jax.experimental.pallas import tpu as pltpu
from jax.experimental.pallas import tpu_sc as plsc


_CLAMP = 5.0

_NC, _NS = 2, 16
_NW = _NC * _NS


def _pick_block(total, target, mult):
    best = None
    d = mult
    while d <= target:
        if total % d == 0:
            best = d
        d += mult
    if best is None:
        best = total
    return best


def _sc_gather_g(qh, kh, dst, src):
    n, d = qh.shape
    e = dst.shape[0]
    per_w = e // _NW
    s = _pick_block(per_w, 248, 8)
    iters = per_w // s
    itersh = iters // 2
    assert iters % 2 == 0
    mesh = plsc.VectorSubcoreMesh(
        core_axis_name="c", subcore_axis_name="s", num_cores=_NC,
        num_subcores=_NS)

    @functools.partial(
        pl.kernel,
        out_type=jax.ShapeDtypeStruct((e, d), jnp.float32),
        mesh=mesh,
        scratch_types=[
            pltpu.VMEM((s,), jnp.int32),
            pltpu.VMEM((s,), jnp.int32),
            pltpu.VMEM((s,), jnp.int32),
            pltpu.VMEM((s,), jnp.int32),
            pltpu.VMEM((2, s, d), jnp.float32),
            pltpu.VMEM((2, s, d), jnp.float32),
        ] + [pltpu.SemaphoreType.DMA] * 8,
    )
    def k(qh_hbm, kh_hbm, dst_hbm, src_hbm, out_hbm, dbufa, dbufb, sbufa,
          sbufb, qbuf2, kbuf2, semq0, semq1, semk0, semk1, semi0, semi1,
          sems0, sems1):
        wid = lax.axis_index("s") * _NC + lax.axis_index("c")
        base0 = wid * per_w
        dbufs = (dbufa, dbufb)
        sbufs = (sbufa, sbufb)
        semq = (semq0, semq1)
        semk = (semk0, semk1)
        semi = (semi0, semi1)
        sems = (sems0, sems1)

        def idx_issue(slot, base):
            pltpu.async_copy(dst_hbm.at[pl.ds(base, s)], dbufs[slot],
                             semi[slot])
            pltpu.async_copy(src_hbm.at[pl.ds(base, s)], sbufs[slot],
                             sems[slot])

        def idx_wait(slot, base):
            pltpu.make_async_copy(dst_hbm.at[pl.ds(base, s)],
                                  dbufs[slot], semi[slot]).wait()
            pltpu.make_async_copy(src_hbm.at[pl.ds(base, s)],
                                  sbufs[slot], sems[slot]).wait()

        def qk_issue(slot):
            pltpu.async_copy(qh_hbm.at[dbufs[slot]], qbuf2.at[slot],
                             semq[slot])
            pltpu.async_copy(kh_hbm.at[sbufs[slot]], kbuf2.at[slot],
                             semk[slot])

        def qk_wait(slot):
            pltpu.make_async_copy(qh_hbm.at[dbufs[slot]], qbuf2.at[slot],
                                  semq[slot]).wait()
            pltpu.make_async_copy(kh_hbm.at[sbufs[slot]], kbuf2.at[slot],
                                  semk[slot]).wait()

        def addstore(slot, base):
            qb = qbuf2.at[slot]
            kb = kbuf2.at[slot]

            @plsc.parallel_loop(0, s, step=1, unroll=4)
            def _(ei):
                for j in range(d // 16):
                    sl = pl.ds(j * 16, 16)
                    qb[ei, sl] += kb[ei, sl]

            pltpu.sync_copy(qb, out_hbm.at[pl.ds(base, s)])

        idx_issue(0, base0)
        idx_wait(0, base0)
        qk_issue(0)
        idx_issue(1, base0 + s)

        def body(kk, carry):
            basea = base0 + (2 * kk) * s
            baseb = basea + s
            basen = baseb + s
            idx_wait(1, baseb)
            qk_issue(1)
            qk_wait(0)
            addstore(0, basea)

            @pl.when(2 * kk + 2 < iters)
            def _():
                idx_issue(0, basen)

            qk_wait(1)

            @pl.when(2 * kk + 2 < iters)
            def _():
                idx_wait(0, basen)
                qk_issue(0)

            addstore(1, baseb)

            @pl.when(2 * kk + 3 < iters)
            def _():
                idx_issue(1, basen + s)

            return carry

        lax.fori_loop(0, itersh, body, 0)

    return k(qh, kh, dst, src)


def _sc_scatter_nagg(vh, wpad, dst, src):
    n, d = vh.shape
    e = dst.shape[0]
    nh = d // 16
    n8 = -(-n // 8)
    smax = (1_750_000 - n * d - (n8 + 512) * 128) // (
        _NS * (2 * (d + 16 + 2) + 128 + 1))
    s = _pick_block(e, min(smax, 512), 16)
    nblke = e // s
    cz = _pick_block(n, s, 8)
    nblk = n // cz
    reps = -(-nblk // _NS)
    n8p = -(-n8 // cz) * cz
    nblk2 = n8p // cz
    mesh = plsc.VectorSubcoreMesh(
        core_axis_name="c", subcore_axis_name="s", num_cores=_NC,
        num_subcores=_NS)

    @functools.partial(
        pl.kernel,
        out_type=(
            jax.ShapeDtypeStruct((_NC, n, d), jnp.float32),
            jax.ShapeDtypeStruct((_NC, n8p, 128), jnp.float32),
        ),
        mesh=mesh,
        scratch_types=[
            pltpu.VMEM((s,), jnp.int32),
            pltpu.VMEM((s,), jnp.int32),
            pltpu.VMEM((s,), jnp.int32),
            pltpu.VMEM((s,), jnp.int32),
            pltpu.VMEM((s,), jnp.int32),
            pltpu.VMEM((2, s, d), jnp.float32),
            pltpu.VMEM((2, s, 16), jnp.float32),
            pltpu.VMEM((s, 128), jnp.float32),
            pltpu.VMEM_SHARED((n, d), jnp.float32),
            pltpu.VMEM_SHARED((n8p, 128), jnp.float32),
        ] + [pltpu.SemaphoreType.DMA] * 8,
    )
    def k(vh_hbm, wpad_hbm, dst_hbm, src_hbm, outv_hbm, outs_hbm, dbufa,
          dbufb, d8buf, sbufa, sbufb, vbuf2, wbuf2, wc, table, tables,
          semi0, semi1, sems0, sems1, semw0, semw1, semv0, semv1):
        cid = lax.axis_index("c")
        sid = lax.axis_index("s")
        wid = sid * _NC + cid
        t_count = (nblke - wid + _NW - 1) // _NW
        zv = jnp.zeros((16,), jnp.float32)
        dbufs = (dbufa, dbufb)
        sbufs = (sbufa, sbufb)
        semi = (semi0, semi1)
        sems = (sems0, sems1)
        semw = (semw0, semw1)
        semv = (semv0, semv1)

        vb0 = vbuf2.at[0]

        @plsc.parallel_loop(0, s, step=1, unroll=4)
        def _(r):
            for j in range(d // 16):
                vb0[r, pl.ds(j * 16, 16)] = zv
                wc[r, pl.ds(j * 16, 16)] = zv

        for rep in range(reps):
            b = sid + rep * _NS

            @pl.when(b < nblk)
            def _():
                pltpu.sync_copy(vb0.at[pl.ds(0, cz)],
                                table.at[pl.ds(b * cz, cz)])
        for rep in range(-(-nblk2 // _NS)):
            b = sid + rep * _NS

            @pl.when(b < nblk2)
            def _():
                pltpu.sync_copy(vb0.at[pl.ds(0, cz)],
                                tables.at[pl.ds(b * cz, cz)])
        plsc.subcore_barrier()

        def idx_issue(slot, base):
            pltpu.async_copy(dst_hbm.at[pl.ds(base, s)], dbufs[slot],
                             semi[slot])
            pltpu.async_copy(src_hbm.at[pl.ds(base, s)], sbufs[slot],
                             sems[slot])
            pltpu.async_copy(wpad_hbm.at[pl.ds(base, s)], wbuf2.at[slot],
                             semw[slot])

        def idx_wait(slot, base):
            pltpu.make_async_copy(dst_hbm.at[pl.ds(base, s)],
                                  dbufs[slot], semi[slot]).wait()
            pltpu.make_async_copy(src_hbm.at[pl.ds(base, s)],
                                  sbufs[slot], sems[slot]).wait()
            pltpu.make_async_copy(wpad_hbm.at[pl.ds(base, s)],
                                  wbuf2.at[slot], semw[slot]).wait()

        def v_issue(slot):
            pltpu.async_copy(vh_hbm.at[sbufs[slot]], vbuf2.at[slot],
                             semv[slot])

        def v_wait(slot):
            pltpu.make_async_copy(vh_hbm.at[sbufs[slot]], vbuf2.at[slot],
                                  semv[slot]).wait()

        def wc_build(slot):
            db = dbufs[slot]
            wb = wbuf2.at[slot]

            @plsc.parallel_loop(0, s // 16, step=1, unroll=1)
            def _(g):
                dv = db[pl.ds(g * 16, 16)]
                d8buf[pl.ds(g * 16, 16)] = lax.shift_right_logical(dv, 3)
                kv = lax.shift_left(dv & 7, 4)
                for l in range(16):
                    ei = g * 16 + l
                    ko = pl.multiple_of(kv[l], 16)
                    wc[ei, pl.ds(ko, 16)] = wb[ei, :]

        def rowmul(slot):
            vb = vbuf2.at[slot]
            wb = wbuf2.at[slot]

            @plsc.parallel_loop(0, s, step=1, unroll=4)
            def _(ei):
                wrow = wb[ei, :]
                for h in range(nh):
                    sl = pl.ds(h * 16, 16)
                    wv = jnp.full((16,), wrow[h])
                    vb[ei, sl] = vb[ei, sl] * wv

        def scatter_erase(slot, erase):
            db = dbufs[slot]
            pltpu.sync_copy(vbuf2.at[slot], table.at[db], add=True)
            pltpu.sync_copy(wc, tables.at[d8buf], add=True)
            if erase:
                @plsc.parallel_loop(0, s // 16, step=1, unroll=1)
                def _(g):
                    dv = db[pl.ds(g * 16, 16)]
                    kv = lax.shift_left(dv & 7, 4)
                    for l in range(16):
                        ko = pl.multiple_of(kv[l], 16)
                        wc[g * 16 + l, pl.ds(ko, 16)] = zv

        def cbase(t):
            return (wid + t * _NW) * s

        idx_issue(0, cbase(0))
        idx_wait(0, cbase(0))
        v_issue(0)

        @pl.when(t_count > 1)
        def _():
            idx_issue(1, cbase(1))

        def body(kk, carry):
            basea = cbase(2 * kk)
            baseb = basea + _NW * s
            basen = baseb + _NW * s
            wc_build(0)
            v_wait(0)
            idx_wait(1, baseb)
            v_issue(1)
            rowmul(0)
            scatter_erase(0, True)

            @pl.when(2 * kk + 2 < t_count)
            def _():
                idx_issue(0, basen)

            wc_build(1)
            v_wait(1)

            @pl.when(2 * kk + 2 < t_count)
            def _():
                idx_wait(0, basen)
                v_issue(0)

            rowmul(1)
            scatter_erase(1, True)

            @pl.when(2 * kk + 3 < t_count)
            def _():
                idx_issue(1, basen + _NW * s)

            return carry

        lax.fori_loop(0, t_count // 2, body, 0)

        @pl.when(t_count % 2 == 1)
        def _():
            wc_build(0)
            v_wait(0)
            rowmul(0)
            scatter_erase(0, False)

        plsc.subcore_barrier()
        for rep in range(reps):
            b = sid + rep * _NS

            @pl.when(b < nblk)
            def _():
                pltpu.sync_copy(table.at[pl.ds(b * cz, cz)],
                                outv_hbm.at[cid, pl.ds(b * cz, cz)])
        for rep in range(-(-nblk2 // _NS)):
            b = sid + rep * _NS

            @pl.when(b < nblk2)
            def _():
                pltpu.sync_copy(tables.at[pl.ds(b * cz, cz)],
                                outs_hbm.at[cid, pl.ds(b * cz, cz)])

    return k(vh, wpad, dst, src), n8p


def _sc_scatter_cagg(conn2raw, wpad, dst, mu_iv, n):
    e, d = conn2raw.shape
    nh = d // 16
    per_w = e // _NW
    smax = (1_700_000 - n * d - 256 * _NS) // (
        _NS * (2 * (d + 16 + 1) + d))
    s = _pick_block(per_w, min(smax, 512), 8)
    iters = per_w // s
    nblk = n // s
    reps = -(-nblk // _NS)
    mesh = plsc.VectorSubcoreMesh(
        core_axis_name="c", subcore_axis_name="s", num_cores=_NC,
        num_subcores=_NS)

    @functools.partial(
        pl.kernel,
        out_type=(
            jax.ShapeDtypeStruct((e, d), jnp.float32),
            jax.ShapeDtypeStruct((_NC, n, d), jnp.float32),
        ),
        mesh=mesh,
        scratch_types=[
            pltpu.VMEM((s,), jnp.int32),
            pltpu.VMEM((s,), jnp.int32),
            pltpu.VMEM((2, s, d), jnp.float32),
            pltpu.VMEM((2, s, 16), jnp.float32),
            pltpu.VMEM((s, d), jnp.float32),
            pltpu.VMEM((2, d), jnp.float32),
            pltpu.VMEM_SHARED((n, d), jnp.float32),
        ] + [pltpu.SemaphoreType.DMA] * 6,
    )
    def k(c2_hbm, wpad_hbm, dst_hbm, muiv_hbm, c2n_hbm, out_hbm, dbufa,
          dbufb, cbuf2, wbuf2, contrib, muiv, table, semi0, semi1, semw0,
          semw1, semc0, semc1):
        cid = lax.axis_index("c")
        sid = lax.axis_index("s")
        wid = sid * _NC + cid
        base0 = wid * per_w
        dbufs = (dbufa, dbufb)
        semi = (semi0, semi1)
        semw = (semw0, semw1)
        semc = (semc0, semc1)

        pltpu.sync_copy(muiv_hbm, muiv)
        mv = [muiv[0, pl.ds(j * 16, 16)] for j in range(nh)]
        vv = [muiv[1, pl.ds(j * 16, 16)] for j in range(nh)]

        @plsc.parallel_loop(0, s, step=1, unroll=4)
        def _(r):
            for j in range(d // 16):
                contrib[r, pl.ds(j * 16, 16)] = jnp.zeros((16,), jnp.float32)

        for rep in range(reps):
            b = sid + rep * _NS

            @pl.when(b < nblk)
            def _():
                pltpu.sync_copy(contrib, table.at[pl.ds(b * s, s)])
        plsc.subcore_barrier()

        def loads_issue(slot, base):
            pltpu.async_copy(dst_hbm.at[pl.ds(base, s)], dbufs[slot],
                             semi[slot])
            pltpu.async_copy(wpad_hbm.at[pl.ds(base, s)], wbuf2.at[slot],
                             semw[slot])
            pltpu.async_copy(c2_hbm.at[pl.ds(base, s)], cbuf2.at[slot],
                             semc[slot])

        def loads_wait(slot, base):
            pltpu.make_async_copy(dst_hbm.at[pl.ds(base, s)],
                                  dbufs[slot], semi[slot]).wait()
            pltpu.make_async_copy(wpad_hbm.at[pl.ds(base, s)],
                                  wbuf2.at[slot], semw[slot]).wait()
            pltpu.make_async_copy(c2_hbm.at[pl.ds(base, s)],
                                  cbuf2.at[slot], semc[slot]).wait()

        def chunk(slot, base):
            cb = cbuf2.at[slot]
            wb = wbuf2.at[slot]

            @plsc.parallel_loop(0, s, step=1, unroll=4)
            def _(ei):
                wrow = wb[ei, :]
                for h in range(nh):
                    sl = pl.ds(h * 16, 16)
                    t = jnp.maximum((cb[ei, sl] - mv[h]) * vv[h], 0.0)
                    cb[ei, sl] = t
                    wv = jnp.full((16,), wrow[h])
                    contrib[ei, sl] = t * wv

            pltpu.sync_copy(cb, c2n_hbm.at[pl.ds(base, s)])
            pltpu.sync_copy(contrib, table.at[dbufs[slot]], add=True)

        loads_issue(0, base0)
        loads_issue(1, base0 + s)

        def body(kk, carry):
            basea = base0 + (2 * kk) * s
            baseb = basea + s
            loads_wait(0, basea)
            chunk(0, basea)

            @pl.when(2 * kk + 2 < iters)
            def _():
                loads_issue(0, baseb + s)

            loads_wait(1, baseb)
            chunk(1, baseb)

            @pl.when(2 * kk + 3 < iters)
            def _():
                loads_issue(1, baseb + 2 * s)

            return carry

        lax.fori_loop(0, iters // 2, body, 0)
        if iters % 2 == 1:
            lastb = base0 + (iters - 1) * s
            loads_wait(0, lastb)
            chunk(0, lastb)
        plsc.subcore_barrier()
        for rep in range(reps):
            b = sid + rep * _NS

            @pl.when(b < nblk)
            def _():
                pltpu.sync_copy(table.at[pl.ds(b * s, s)],
                                out_hbm.at[cid, pl.ds(b * s, s)])

    return k(conn2raw, wpad, dst, mu_iv)


def _qkv_body(x_ref, wt_ref, b_ref, q_ref, k_ref, v_ref):
    d = x_ref.shape[1]
    qkv = (
        jnp.dot(x_ref[...], wt_ref[...], preferred_element_type=jnp.float32)
        + b_ref[...]
    )
    q_ref[...] = qkv[:, :d]
    k_ref[...] = qkv[:, d:2 * d]
    v_ref[...] = qkv[:, 2 * d:]


def _tc_qkv(x, qkv_weight, qkv_bias):
    n, d = x.shape
    bn = _pick_block(n, 2048, 8)
    grid = n // bn
    return pl.pallas_call(
        _qkv_body,
        grid=(grid,),
        in_specs=[
            pl.BlockSpec((bn, d), lambda i: (i, 0)),
            pl.BlockSpec((d, 3 * d), lambda i: (0, 0)),
            pl.BlockSpec((1, 3 * d), lambda i: (0, 0)),
        ],
        out_specs=[pl.BlockSpec((bn, d), lambda i: (i, 0))] * 3,
        out_shape=[jax.ShapeDtypeStruct((n, d), jnp.float32)] * 3,
    )(x, qkv_weight.T, qkv_bias.reshape(1, -1))


def _edge_a_body(g_ref, poly_ref, w1t_ref, b1_ref, w2t_ref, b2_ref, wsc_ref,
                 conn2_ref, wpad_ref, stats_ref):
    i = pl.program_id(0)
    poly = poly_ref[...]
    d = poly.shape[1]
    eh = (
        jnp.dot(poly, w1t_ref[...], preferred_element_type=jnp.float32)
        + b1_ref[...]
    )
    ew = eh[:, :d]
    ebias = eh[:, d:]
    c = g_ref[...] * ew
    c = jnp.sign(c) * jnp.sqrt(jnp.abs(c)) + ebias
    conn = jnp.maximum(c, 0.0)
    conn2 = (
        jnp.dot(conn, w2t_ref[...], preferred_element_type=jnp.float32)
        + b2_ref[...]
        + poly
    )
    conn2_ref[...] = conn2
    s = jnp.dot(conn, wsc_ref[...], preferred_element_type=jnp.float32)
    wpad_ref[...] = jnp.exp(jnp.clip(s, -_CLAMP, _CLAMP))

    s1 = jnp.sum(conn2, axis=0)
    s2 = jnp.sum(conn2 * conn2, axis=0)
    blk = jnp.concatenate(
        [s1[None, :], s2[None, :], jnp.zeros((6, d), jnp.float32)], axis=0
    )

    @pl.when(i == 0)
    def _():
        stats_ref[...] = jnp.zeros_like(stats_ref)

    stats_ref[...] += blk


def _tc_edge_a(g, poly_conn, conn_lin1_w, conn_lin1_b, conn_lin2_w,
               conn_lin2_b, wsc2):
    e, d = poly_conn.shape
    be = _pick_block(e, 4096, 128)
    grid = e // be
    return pl.pallas_call(
        _edge_a_body,
        grid=(grid,),
        in_specs=[
            pl.BlockSpec((be, d), lambda i: (i, 0)),
            pl.BlockSpec((be, d), lambda i: (i, 0)),
            pl.BlockSpec((d, 2 * d), lambda i: (0, 0)),
            pl.BlockSpec((1, 2 * d), lambda i: (0, 0)),
            pl.BlockSpec((d, d), lambda i: (0, 0)),
            pl.BlockSpec((1, d), lambda i: (0, 0)),
            pl.BlockSpec((d, 16), lambda i: (0, 0)),
        ],
        out_specs=[
            pl.BlockSpec((be, d), lambda i: (i, 0)),
            pl.BlockSpec((be, 16), lambda i: (i, 0)),
            pl.BlockSpec((8, d), lambda i: (0, 0)),
        ],
        out_shape=[
            jax.ShapeDtypeStruct((e, d), jnp.float32),
            jax.ShapeDtypeStruct((e, 16), jnp.float32),
            jax.ShapeDtypeStruct((8, d), jnp.float32),
        ],
    )(g, poly_conn, conn_lin1_w.T, conn_lin1_b.reshape(1, -1),
      conn_lin2_w.T, conn_lin2_b.reshape(1, -1), wsc2)


def _node1_body(p0_ref, p1_ref, s0_ref, s1_ref, c0_ref, c1_ref, x_ref,
                sd_ref, dc0_ref, dc1_ref, hres_ref, stats_ref):
    i = pl.program_id(0)
    d = x_ref.shape[1]
    h = d // 16
    naggs = p0_ref[...] + p1_ref[...] + c0_ref[...] + c1_ref[...]
    ssum = s0_ref[:, :h] + s1_ref[:, :h]
    f_idx = lax.broadcasted_iota(jnp.int32, (h, d), 1)
    h_idx = lax.broadcasted_iota(jnp.int32, (h, d), 0)
    expand = (f_idx // 16 == h_idx).astype(jnp.float32)
    den = jnp.dot(ssum, expand, preferred_element_type=jnp.float32)
    agg = naggs / (den + 1e-16)
    nh = agg * (dc0_ref[...] + sd_ref[...] * dc1_ref[...])
    hres = nh + x_ref[...]
    hres_ref[...] = hres

    s1 = jnp.sum(hres, axis=0)
    s2 = jnp.sum(hres * hres, axis=0)
    blk = jnp.concatenate(
        [s1[None, :], s2[None, :], jnp.zeros((6, d), jnp.float32)], axis=0
    )

    @pl.when(i == 0)
    def _():
        stats_ref[...] = jnp.zeros_like(stats_ref)

    stats_ref[...] += blk


def _tc_node1(p0, p1, s0, s1, c0, c1, x, sqrt_deg, dc0, dc1):
    n, d = x.shape
    bn = _pick_block(n, 2048, 8)
    grid = n // bn
    return pl.pallas_call(
        _node1_body,
        grid=(grid,),
        in_specs=[
            pl.BlockSpec((bn, d), lambda i: (i, 0)),
            pl.BlockSpec((bn, d), lambda i: (i, 0)),
            pl.BlockSpec((bn, 16), lambda i: (i, 0)),
            pl.BlockSpec((bn, 16), lambda i: (i, 0)),
            pl.BlockSpec((bn, d), lambda i: (i, 0)),
            pl.BlockSpec((bn, d), lambda i: (i, 0)),
            pl.BlockSpec((bn, d), lambda i: (i, 0)),
            pl.BlockSpec((bn, 1), lambda i: (i, 0)),
            pl.BlockSpec((1, d), lambda i: (0, 0)),
            pl.BlockSpec((1, d), lambda i: (0, 0)),
        ],
        out_specs=[
            pl.BlockSpec((bn, d), lambda i: (i, 0)),
            pl.BlockSpec((8, d), lambda i: (0, 0)),
        ],
        out_shape=[
            jax.ShapeDtypeStruct((n, d), jnp.float32),
            jax.ShapeDtypeStruct((8, d), jnp.float32),
        ],
    )(p0, p1, s0, s1, c0, c1, x, sqrt_deg, dc0, dc1)


def _node2_body(hres_ref, mu_ref, iv_ref, f1t_ref, fb1_ref, f2t_ref, fb2_ref,
                t2_ref, stats_ref):
    i = pl.program_id(0)
    d = hres_ref.shape[1]
    hres = hres_ref[...]
    t = (hres - mu_ref[...]) * iv_ref[...]
    t1 = jnp.maximum(
        jnp.dot(t, f1t_ref[...], preferred_element_type=jnp.float32)
        + fb1_ref[...],
        0.0,
    )
    t2 = (
        jnp.dot(t1, f2t_ref[...], preferred_element_type=jnp.float32)
        + fb2_ref[...]
        + hres
    )
    t2_ref[...] = t2

    s1 = jnp.sum(t2, axis=0)
    s2 = jnp.sum(t2 * t2, axis=0)
    blk = jnp.concatenate(
        [s1[None, :], s2[None, :], jnp.zeros((6, d), jnp.float32)], axis=0
    )

    @pl.when(i == 0)
    def _():
        stats_ref[...] = jnp.zeros_like(stats_ref)

    stats_ref[...] += blk


def _tc_node2(hres, mu, iv, ffn1_w, ffn1_b, ffn2_w, ffn2_b):
    n, d = hres.shape
    bn = _pick_block(n, 2048, 8)
    grid = n // bn
    return pl.pallas_call(
        _node2_body,
        grid=(grid,),
        in_specs=[
            pl.BlockSpec((bn, d), lambda i: (i, 0)),
            pl.BlockSpec((1, d), lambda i: (0, 0)),
            pl.BlockSpec((1, d), lambda i: (0, 0)),
            pl.BlockSpec((d, 2 * d), lambda i: (0, 0)),
            pl.BlockSpec((1, 2 * d), lambda i: (0, 0)),
            pl.BlockSpec((2 * d, d), lambda i: (0, 0)),
            pl.BlockSpec((1, d), lambda i: (0, 0)),
        ],
        out_specs=[
            pl.BlockSpec((bn, d), lambda i: (i, 0)),
            pl.BlockSpec((8, d), lambda i: (0, 0)),
        ],
        out_shape=[
            jax.ShapeDtypeStruct((n, d), jnp.float32),
            jax.ShapeDtypeStruct((8, d), jnp.float32),
        ],
    )(hres, mu, iv, ffn1_w.T, ffn1_b.reshape(1, -1), ffn2_w.T,
      ffn2_b.reshape(1, -1))


def _node3_body(t2_ref, mu_ref, iv_ref, o_ref):
    o_ref[...] = (t2_ref[...] - mu_ref[...]) * iv_ref[...]


def _tc_node3(t2, mu, iv):
    n, d = t2.shape
    bn = _pick_block(n, 2048, 8)
    grid = n // bn
    return pl.pallas_call(
        _node3_body,
        grid=(grid,),
        in_specs=[
            pl.BlockSpec((bn, d), lambda i: (i, 0)),
            pl.BlockSpec((1, d), lambda i: (0, 0)),
            pl.BlockSpec((1, d), lambda i: (0, 0)),
        ],
        out_specs=pl.BlockSpec((bn, d), lambda i: (i, 0)),
        out_shape=jax.ShapeDtypeStruct((n, d), jnp.float32),
    )(t2, mu, iv)


def _stats_to_mu_inv(stats, count):
    s1 = stats[0]
    s2 = stats[1]
    mu = s1 / count
    var = s2 / count - mu * mu
    iv = 1.0 / jnp.sqrt(var + 1e-5)
    return mu.reshape(1, -1), iv.reshape(1, -1)


def kernel(x, edge_index, poly_conn, sqrt_deg, qkv_weight, qkv_bias,
           conn_lin1_w, conn_lin1_b, Wscore, conn_lin2_w, conn_lin2_b,
           deg_coef, ffn1_w, ffn1_b, ffn2_w, ffn2_b):
    n, d = x.shape
    e = poly_conn.shape[0]
    h = d // 16

    dst = edge_index[0]
    src = edge_index[1]

    wsq = Wscore[:, :, 0]
    sel = jnp.zeros((h, 16), jnp.float32).at[:, :h].set(jnp.eye(h))
    wsc2 = jnp.einsum('dh,hc->hdc', wsq, sel).reshape(d, 16)

    qh, kh, vh = _tc_qkv(x, qkv_weight, qkv_bias)

    g = _sc_gather_g(qh, kh, dst, src)

    conn2raw, wpad, estats = _tc_edge_a(
        g, poly_conn, conn_lin1_w, conn_lin1_b, conn_lin2_w, conn_lin2_b,
        wsc2)
    emu, eiv = _stats_to_mu_inv(estats, float(e))

    (part_v, part_s), n8p = _sc_scatter_nagg(vh, wpad, dst, src)
    mu_iv = jnp.concatenate([emu, eiv], axis=0)
    conn2n, part_c = _sc_scatter_cagg(conn2raw, wpad, dst, mu_iv, n)
    ssums = part_s.reshape(_NC, n8p * 8, 16)[:, :n, :]

    dc0 = deg_coef[0, :, 0].reshape(1, d)
    dc1 = deg_coef[0, :, 1].reshape(1, d)
    hres, nstats1 = _tc_node1(part_v[0], part_v[1], ssums[0], ssums[1],
                              part_c[0], part_c[1], x, sqrt_deg, dc0, dc1)
    mu1, iv1 = _stats_to_mu_inv(nstats1, float(n))
    t2, nstats2 = _tc_node2(hres, mu1, iv1, ffn1_w, ffn1_b, ffn2_w, ffn2_b)
    mu2, iv2 = _stats_to_mu_inv(nstats2, float(n))
    nh = _tc_node3(t2, mu2, iv2)
    return nh, conn2n
